# Initial kernel scaffold; baseline (speedup 1.0000x reference)
import jax
import jax.numpy as jnp
from jax.experimental import pallas as pl


def kernel(features, means3D, cov3D, opacities):
    raise NotImplementedError("write your pallas kernel here")


# speedup vs baseline: 7.6500x; 7.6500x over previous
import functools

import jax
import jax.numpy as jnp
from jax import lax
from jax.experimental import pallas as pl

H = 200
W = 200
SH = 200.0 / 100.0
SW = 200.0 / 100.0
THRESHOLD = 0.05
TH = 8


def _splat_kernel(feat_ref, mean_ref, cov_ref, opac_ref, img_ref, cnt_ref):
    t = pl.program_id(1)

    mean = mean_ref[0]
    cov = cov_ref[0]
    op = opac_ref[0]

    u = H / 2.0 - SH * mean[:, 1:2]
    v = W / 2.0 - SW * mean[:, 0:1]
    A = SH * SH * cov[:, 3:4] + 0.3
    C = SW * SW * cov[:, 0:1] + 0.3
    B = SH * SW * cov[:, 1:2]
    det = A * C - B * B
    det = jnp.where(det <= 1e-8, 1e-8, det)
    conA = C / det
    conB = -B / det
    conC = A / det

    maskf = (op > THRESHOLD).astype(jnp.float32)
    opm = op * maskf
    cnt_ref[0] = jnp.sum(maskf, axis=0, keepdims=True)

    xs = lax.broadcasted_iota(jnp.int32, (1, W), 1).astype(jnp.float32) + 0.5
    dv = xs - v
    hterm = -0.5 * (conC * dv) * dv
    cbdv = conB * dv

    feat = feat_ref[0]
    y0 = (t * TH).astype(jnp.float32) + 0.5
    for r in range(TH):
        du = (y0 + float(r)) - u
        aterm = -0.5 * (conA * du) * du
        power = (aterm + hterm) - du * cbdv
        power = jnp.minimum(power, 0.0)
        alpha = jnp.minimum(opm * jnp.exp(power), 0.99)
        row = lax.dot_general(feat, alpha, (((0,), (0,)), ((), ())),
                              preferred_element_type=jnp.float32)
        img_ref[0, :, r, :] = row


@functools.partial(jax.jit, static_argnames=())
def kernel(features, means3D, cov3D, opacities):
    b, p, d = features.shape
    n_tiles = H // TH
    grid = (b, n_tiles)
    img, counts = pl.pallas_call(
        _splat_kernel,
        grid=grid,
        in_specs=[
            pl.BlockSpec((1, p, d), lambda bi, ti: (bi, 0, 0)),
            pl.BlockSpec((1, p, 3), lambda bi, ti: (bi, 0, 0)),
            pl.BlockSpec((1, p, 6), lambda bi, ti: (bi, 0, 0)),
            pl.BlockSpec((1, p, 1), lambda bi, ti: (bi, 0, 0)),
        ],
        out_specs=[
            pl.BlockSpec((1, d, TH, W), lambda bi, ti: (bi, 0, ti, 0)),
            pl.BlockSpec((1, 1, 1), lambda bi, ti: (bi, 0, 0)),
        ],
        out_shape=[
            jax.ShapeDtypeStruct((b, d, H, W), jnp.float32),
            jax.ShapeDtypeStruct((b, 1, 1), jnp.float32),
        ],
    )(features, means3D, cov3D, opacities)
    return img, jnp.mean(counts)
